# baseline (speedup 1.0000x reference)
import jax
import jax.numpy as jnp
from jax.experimental import pallas as pl
from jax.experimental.pallas import tpu as pltpu
from jax.experimental.pallas import tpu_sc as plsc


def _sc_gather(feat, idx_flat, window):
    m = idx_flat.shape[1]
    c = feat.shape[1]
    mesh = plsc.VectorSubcoreMesh(core_axis_name="core", subcore_axis_name="subcore")

    @pl.kernel(out_type=jax.ShapeDtypeStruct((m, c), feat.dtype), mesh=mesh)
    def gather_kernel(x_hbm, i_hbm, o_hbm):
        def body(i_vmem, o_vmem):
            pltpu.sync_copy(x_hbm.at[i_vmem.at[0]], o_vmem)

        pltpu.emit_pipeline(
            body,
            grid=(m // window,),
            in_specs=[pl.BlockSpec((1, window), lambda i: (0, i))],
            out_specs=[pl.BlockSpec((window, c), lambda i: (i, 0))],
            core_axis_name=("core", "subcore"),
            dimension_semantics=(pltpu.PARALLEL,),
        )(i_hbm, o_hbm)

    return gather_kernel(feat, idx_flat)


def _tc_matmul(gathered, weightnet, additional, block_n):
    n, k, c_in = gathered.shape
    c_mid = weightnet.shape[3]
    c_add = additional.shape[3]
    c_tot = c_in + c_add

    def body(g_ref, w_ref, a_ref, o_ref):
        g = g_ref[...]
        w = w_ref[0]
        a = a_ref[0]
        og = jax.lax.dot_general(
            g, w, (((1,), (1,)), ((0,), (0,))), preferred_element_type=jnp.float32
        )
        oa = jax.lax.dot_general(
            a, w, (((1,), (1,)), ((0,), (0,))), preferred_element_type=jnp.float32
        )
        o_ref[:, :c_in, :] = og
        o_ref[:, c_in:, :] = oa

    return pl.pallas_call(
        body,
        grid=(n // block_n,),
        in_specs=[
            pl.BlockSpec((block_n, k, c_in), lambda i: (i, 0, 0)),
            pl.BlockSpec((1, block_n, k, c_mid), lambda i: (0, i, 0, 0)),
            pl.BlockSpec((1, block_n, k, c_add), lambda i: (0, i, 0, 0)),
        ],
        out_specs=pl.BlockSpec((block_n, c_tot, c_mid), lambda i: (i, 0, 0)),
        out_shape=jax.ShapeDtypeStruct((n, c_tot, c_mid), jnp.float32),
    )(gathered, weightnet, additional)


def kernel(input_features, neighbor_inds, weightnet, additional_features):
    b, n, c_in = input_features.shape
    k = neighbor_inds.shape[2]
    c_mid = weightnet.shape[3]
    c_add = additional_features.shape[3]

    feat = input_features.reshape(n, c_in)
    idx_flat = neighbor_inds.reshape(1, n * k)
    gathered = _sc_gather(feat, idx_flat, window=128)

    out = _tc_matmul(
        gathered.reshape(n, k, c_in),
        weightnet,
        additional_features,
        block_n=100,
    )
    return out.reshape(b, n, (c_in + c_add) * c_mid)


# speedup vs baseline: 1.0323x; 1.0323x over previous
import jax
import jax.numpy as jnp
from jax.experimental import pallas as pl
from jax.experimental.pallas import tpu as pltpu
from jax.experimental.pallas import tpu_sc as plsc


def _sc_gather(feat, idx_flat, window):
    m = idx_flat.shape[1]
    c = feat.shape[1]
    mesh = plsc.VectorSubcoreMesh(core_axis_name="core", subcore_axis_name="subcore")

    @pl.kernel(out_type=jax.ShapeDtypeStruct((m, c), feat.dtype), mesh=mesh)
    def gather_kernel(x_hbm, i_hbm, o_hbm):
        def body(i_vmem, o_vmem):
            pltpu.sync_copy(x_hbm.at[i_vmem.at[0]], o_vmem)

        pltpu.emit_pipeline(
            body,
            grid=(m // window,),
            in_specs=[pl.BlockSpec((1, window), lambda i: (0, i))],
            out_specs=[pl.BlockSpec((window, c), lambda i: (i, 0))],
            core_axis_name=("core", "subcore"),
            dimension_semantics=(pltpu.PARALLEL,),
        )(i_hbm, o_hbm)

    return gather_kernel(feat, idx_flat)


def _tc_matmul(gathered, weightnet, additional, block_n):
    n, k, c_in = gathered.shape
    c_mid = weightnet.shape[2]
    c_add = additional.shape[2]

    def body(g_ref, w_ref, a_ref, og_ref, oa_ref):
        g = g_ref[...]
        w = w_ref[...]
        a = a_ref[...]
        og_ref[...] = jax.lax.dot_general(
            w, g, (((1,), (1,)), ((0,), (0,))), preferred_element_type=jnp.float32
        )
        oa_ref[...] = jax.lax.dot_general(
            w, a, (((1,), (1,)), ((0,), (0,))), preferred_element_type=jnp.float32
        )

    return pl.pallas_call(
        body,
        grid=(n // block_n,),
        in_specs=[
            pl.BlockSpec((block_n, k, c_in), lambda i: (i, 0, 0)),
            pl.BlockSpec((block_n, k, c_mid), lambda i: (i, 0, 0)),
            pl.BlockSpec((block_n, k, c_add), lambda i: (i, 0, 0)),
        ],
        out_specs=[
            pl.BlockSpec((block_n, c_mid, c_in), lambda i: (i, 0, 0)),
            pl.BlockSpec((block_n, c_mid, c_add), lambda i: (i, 0, 0)),
        ],
        out_shape=[
            jax.ShapeDtypeStruct((n, c_mid, c_in), jnp.float32),
            jax.ShapeDtypeStruct((n, c_mid, c_add), jnp.float32),
        ],
    )(gathered, weightnet, additional)


def kernel(input_features, neighbor_inds, weightnet, additional_features):
    b, n, c_in = input_features.shape
    k = neighbor_inds.shape[2]
    c_mid = weightnet.shape[3]
    c_add = additional_features.shape[3]

    feat = input_features.reshape(n, c_in)
    idx_flat = neighbor_inds.reshape(1, n * k)
    gathered = _sc_gather(feat, idx_flat, window=128)

    out_g, out_a = _tc_matmul(
        gathered.reshape(n, k, c_in),
        weightnet.reshape(n, k, c_mid),
        additional_features.reshape(n, k, c_add),
        block_n=200,
    )
    out = jnp.concatenate([out_g, out_a], axis=2)
    out = jnp.transpose(out, (0, 2, 1)).reshape(b, n, (c_in + c_add) * c_mid)
    return out


# speedup vs baseline: 1.7111x; 1.6575x over previous
import jax
import jax.numpy as jnp
from jax.experimental import pallas as pl
from jax.experimental.pallas import tpu as pltpu
from jax.experimental.pallas import tpu_sc as plsc


def _sc_gather(feat, idx_flat, window):
    m = idx_flat.shape[1]
    c = feat.shape[1]
    mesh = plsc.VectorSubcoreMesh(core_axis_name="core", subcore_axis_name="subcore")

    @pl.kernel(out_type=jax.ShapeDtypeStruct((m, c), feat.dtype), mesh=mesh)
    def gather_kernel(x_hbm, i_hbm, o_hbm):
        def body(i_vmem, o_vmem):
            pltpu.sync_copy(x_hbm.at[i_vmem.at[0]], o_vmem)

        pltpu.emit_pipeline(
            body,
            grid=(m // window,),
            in_specs=[pl.BlockSpec((1, window), lambda i: (0, i))],
            out_specs=[pl.BlockSpec((window, c), lambda i: (i, 0))],
            core_axis_name=("core", "subcore"),
            dimension_semantics=(pltpu.PARALLEL,),
        )(i_hbm, o_hbm)

    return gather_kernel(feat, idx_flat)


def _tc_matmul(gathered, weightnet, additional, block_n):
    n, k, c_in = gathered.shape
    c_mid = weightnet.shape[2]
    c_add = additional.shape[2]

    def body(g_ref, w_ref, a_ref, o_ref):
        g = g_ref[...]
        w = w_ref[...]
        a = a_ref[...]
        og = jax.lax.dot_general(
            g, w, (((1,), (1,)), ((0,), (0,))), preferred_element_type=jnp.float32
        )
        oa = jax.lax.dot_general(
            a, w, (((1,), (1,)), ((0,), (0,))), preferred_element_type=jnp.float32
        )
        o_ref[:, : c_in * c_mid] = og.reshape(block_n, c_in * c_mid)
        o_ref[:, c_in * c_mid :] = oa.reshape(block_n, c_add * c_mid)

    return pl.pallas_call(
        body,
        grid=(n // block_n,),
        in_specs=[
            pl.BlockSpec((block_n, k, c_in), lambda i: (i, 0, 0)),
            pl.BlockSpec((block_n, k, c_mid), lambda i: (i, 0, 0)),
            pl.BlockSpec((block_n, k, c_add), lambda i: (i, 0, 0)),
        ],
        out_specs=pl.BlockSpec((block_n, (c_in + c_add) * c_mid), lambda i: (i, 0)),
        out_shape=jax.ShapeDtypeStruct((n, (c_in + c_add) * c_mid), jnp.float32),
    )(gathered, weightnet, additional)


def kernel(input_features, neighbor_inds, weightnet, additional_features):
    b, n, c_in = input_features.shape
    k = neighbor_inds.shape[2]
    c_mid = weightnet.shape[3]
    c_add = additional_features.shape[3]

    feat = input_features.reshape(n, c_in)
    idx_flat = neighbor_inds.reshape(1, n * k)
    gathered = _sc_gather(feat, idx_flat, window=128)

    out = _tc_matmul(
        gathered.reshape(n, k, c_in),
        weightnet.reshape(n, k, c_mid),
        additional_features.reshape(n, k, c_add),
        block_n=200,
    )
    return out.reshape(b, n, (c_in + c_add) * c_mid)
